# baseline (speedup 1.0000x reference)
import functools

import jax
import jax.numpy as jnp
from jax import lax
from jax.experimental import pallas as pl
from jax.experimental.pallas import tpu as pltpu
from jax.experimental.pallas import tpu_sc as plsc

TID_DIM = 32
N_TABLE = 288
N_TOTAL = 64 * 12 * 5000
NW = 32
PER_W = N_TOTAL // NW
CHUNK = 240
NCHUNK = PER_W // CHUNK

_mesh = plsc.VectorSubcoreMesh(core_axis_name="c", subcore_axis_name="s")


@functools.partial(
    pl.kernel,
    mesh=_mesh,
    out_type=jax.ShapeDtypeStruct((N_TOTAL, TID_DIM), jnp.float32),
    compiler_params=pltpu.CompilerParams(
        use_tc_tiling_on_sc=True, needs_layout_passes=False
    ),
    scratch_types=[
        pltpu.VMEM((CHUNK,), jnp.int32),
        pltpu.VMEM((CHUNK,), jnp.int32),
        pltpu.VMEM((CHUNK, 128), jnp.float32),
        pltpu.VMEM((CHUNK, 128), jnp.float32),
        pltpu.VMEM((CHUNK, TID_DIM), jnp.float32),
        pltpu.VMEM((CHUNK, TID_DIM), jnp.float32),
        pltpu.SemaphoreType.DMA,
        pltpu.SemaphoreType.DMA,
        pltpu.SemaphoreType.DMA,
    ],
)
def _gather_kernel(table_hbm, idx_hbm, out_hbm, idx0_v, idx1_v,
                   wide0_v, wide1_v, rows0_v, rows1_v, sem0, sem1, gsem):
    wid = lax.axis_index("s") * 2 + lax.axis_index("c")
    w_base = wid * PER_W
    idx_bufs = (idx0_v, idx1_v)
    wide = (wide0_v, wide1_v)
    rows = (rows0_v, rows1_v)
    sems = (sem0, sem1)

    def chunk_body(g, carry):
        for b in range(2):
            j = g * 2 + b
            base = w_base + j * CHUNK

            pltpu.sync_copy(idx_hbm.at[pl.ds(base, CHUNK)], idx_bufs[b])
            gcp = pltpu.async_copy(
                table_hbm.at[idx_bufs[b]], wide[b], gsem
            )

            @pl.when(g > 0)
            def _wait_prev_store():
                pltpu.make_async_copy(
                    rows[b], out_hbm.at[pl.ds(0, CHUNK)], sems[b]
                ).wait()

            gcp.wait()

            @plsc.parallel_loop(0, CHUNK, unroll=8)
            def narrow_body(r):
                rows[b][r, pl.ds(0, 16)] = wide[b][r, pl.ds(0, 16)]
                rows[b][r, pl.ds(16, 16)] = wide[b][r, pl.ds(16, 16)]

            pltpu.async_copy(
                rows[b], out_hbm.at[pl.ds(base, CHUNK)], sems[b]
            )
        return carry

    lax.fori_loop(0, NCHUNK // 2, chunk_body, 0)

    for b in range(2):
        pltpu.make_async_copy(
            rows[b], out_hbm.at[pl.ds(0, CHUNK)], sems[b]
        ).wait()


def kernel(tid, node_emb_in, node_emb_out, tid_table, adp_emb):
    idx = tid.reshape(-1).astype(jnp.int32)
    table_pad = jnp.pad(tid_table, ((0, 0), (0, 128 - TID_DIM)))
    flat = _gather_kernel(table_pad, idx)
    x_tid = flat.reshape(tid.shape + (TID_DIM,))
    return (node_emb_in, node_emb_out, x_tid, adp_emb)
